# baseline (speedup 1.0000x reference)
import functools

import jax
import jax.numpy as jnp
from jax import lax
from jax.experimental import pallas as pl
from jax.experimental.pallas import tpu as pltpu
from jax.experimental.pallas import tpu_sc as plsc

N_NODES = 10000
N_EDGES = 320000
D = 128
DH = 64

NC, NS, L = 2, 16, 16
NW = NC * NS
K = 128
BPT = 80
EP = NW * BPT * K
N_PAD = 10240
NR = N_PAD // NS


def _sc_mesh():
    return plsc.VectorSubcoreMesh(core_axis_name="c", subcore_axis_name="s")


@functools.partial(
    pl.kernel,
    out_type=jax.ShapeDtypeStruct((NC, N_PAD), jnp.float32),
    mesh=_sc_mesh(),
    scratch_types=[
        pltpu.VMEM((BPT, K), jnp.int32),
        pltpu.VMEM((BPT, K), jnp.float32),
        pltpu.VMEM((NR,), jnp.float32),
        pltpu.VMEM_SHARED((N_PAD,), jnp.float32),
    ],
)
def _sc_degree(dst_hbm, w_hbm, out_hbm, didx_all, w_all, z_v, deg_sh):
    c = lax.axis_index("c")
    s = lax.axis_index("s")
    wid = s * NC + c

    pltpu.sync_copy(dst_hbm.at[pl.ds(wid * BPT, BPT)], didx_all)
    pltpu.sync_copy(w_hbm.at[pl.ds(wid * BPT, BPT)], w_all)

    zero = jnp.zeros((L,), jnp.float32)
    for j in range(NR // L):
        z_v[pl.ds(j * L, L)] = zero
    pltpu.sync_copy(z_v, deg_sh.at[pl.ds(s * NR, NR)])
    plsc.subcore_barrier()

    def body(b, carry):
        pltpu.sync_copy(w_all.at[b], deg_sh.at[didx_all.at[b]], add=True)
        return carry

    lax.fori_loop(0, BPT, body, 0)
    plsc.subcore_barrier()
    pltpu.sync_copy(deg_sh.at[pl.ds(s * NR, NR)], out_hbm.at[c, pl.ds(s * NR, NR)])


@functools.partial(
    pl.kernel,
    out_type=(
        jax.ShapeDtypeStruct((NC, N_PAD, DH), jnp.float32),
        jax.ShapeDtypeStruct((NC, N_PAD, DH), jnp.float32),
    ),
    mesh=_sc_mesh(),
    scratch_types=[
        pltpu.VMEM((BPT, K), jnp.int32),
        pltpu.VMEM((BPT, K), jnp.int32),
        pltpu.VMEM((BPT, K), jnp.float32),
        pltpu.VMEM((K, DH), jnp.bfloat16),
        pltpu.VMEM((K, DH), jnp.bfloat16),
        pltpu.VMEM((K, DH), jnp.bfloat16),
        pltpu.VMEM((K, DH), jnp.float32),
        pltpu.VMEM((K, DH), jnp.float32),
        pltpu.VMEM((K, DH), jnp.float32),
        pltpu.VMEM_SHARED((N_PAD, DH), jnp.bfloat16),
        pltpu.VMEM_SHARED((N_PAD, DH), jnp.float32),
        pltpu.SemaphoreType.DMA,
        pltpu.SemaphoreType.DMA,
        pltpu.SemaphoreType.DMA,
        pltpu.SemaphoreType.DMA,
        pltpu.SemaphoreType.DMA,
        pltpu.SemaphoreType.DMA,
    ],
    compiler_params=pltpu.CompilerParams(use_tc_tiling_on_sc=False,
                                         needs_layout_passes=False),
)
def _sc_edge_pass(ya_hbm, yb_hbm, src_hbm, dst_hbm, w_hbm, outa_hbm, outb_hbm,
                  sidx_all, didx_all, w_all, rb0, rb1, rb2, rf0, rf1, rf2,
                  y_sh, acc_sh, g0, g1, g2, s0, s1, s2):
    c = lax.axis_index("c")
    s = lax.axis_index("s")
    wid = s * NC + c
    rbf = (rb0, rb1, rb2)
    rf = (rf0, rf1, rf2)
    gsem = (g0, g1, g2)
    ssem = (s0, s1, s2)

    pltpu.sync_copy(src_hbm.at[pl.ds(wid * BPT, BPT)], sidx_all)
    pltpu.sync_copy(dst_hbm.at[pl.ds(wid * BPT, BPT)], didx_all)
    pltpu.sync_copy(w_hbm.at[pl.ds(wid * BPT, BPT)], w_all)

    zero = jnp.zeros((L,), jnp.float32)

    def gather(b, p):
        pltpu.async_copy(y_sh.at[sidx_all.at[b]], rbf[p], gsem[p])

    def gather_wait(b, p):
        pltpu.make_async_copy(y_sh.at[sidx_all.at[b]], rbf[p], gsem[p]).wait()

    def scatter(b, p):
        pltpu.async_copy(rf[p], acc_sh.at[didx_all.at[b]], ssem[p], add=True)

    def scatter_wait(b, p):
        pltpu.make_async_copy(rf[p], acc_sh.at[didx_all.at[b]], ssem[p]).wait()

    himask = jnp.full((L,), -65536, jnp.int32)

    def scale(b, p):
        def g_body(g, carry):
            wg = w_all[b, pl.ds(g * L, L)]
            for i in range(L):
                wk = wg[i]
                k = g * L + i
                for j in range(DH // (2 * L)):
                    packed = plsc.bitcast(rbf[p][k, pl.ds(j * 2 * L, 2 * L)],
                                          jnp.int32)
                    lo = plsc.bitcast(lax.shift_left(packed, 16), jnp.float32)
                    hi = plsc.bitcast(lax.bitwise_and(packed, himask),
                                      jnp.float32)
                    rf[p][k, pl.ds(j * 2 * L, L)] = lo * wk
                    rf[p][k, pl.ds(j * 2 * L + L, L)] = hi * wk
            return carry

        lax.fori_loop(0, K // L, g_body, 0)

    for y_hbm, out_hbm in ((ya_hbm, outa_hbm), (yb_hbm, outb_hbm)):
        pltpu.sync_copy(y_hbm.at[pl.ds(s * NR, NR)], y_sh.at[pl.ds(s * NR, NR)])

        def zero_rows(r, carry):
            for j in range(DH // L):
                rf0[r, pl.ds(j * L, L)] = zero
            return carry

        lax.fori_loop(0, K, zero_rows, 0)
        for t in range(NR // K):
            pltpu.sync_copy(rf0, acc_sh.at[pl.ds(s * NR + t * K, K)])
        plsc.subcore_barrier()

        gather(0, 0)
        gather(1, 1)

        def body(i, carry):
            for u in range(3):
                b = 3 * i + u
                p = u
                gather_wait(b, p)

                @pl.when(b >= 3)
                def _():
                    scatter_wait(b - 3, p)

                scale(b, p)
                scatter(b, p)

                @pl.when(b + 2 < BPT)
                def _():
                    gather(b + 2, (u + 2) % 3)
            return carry

        n_main = BPT // 3 - 1
        lax.fori_loop(0, n_main, body, 0)

        for b in range(3 * n_main, BPT):
            p = b % 3
            gather_wait(b, p)
            scatter_wait(b - 3, p)
            scale(b, p)
            scatter(b, p)
            if b + 2 < BPT:
                gather(b + 2, (b + 2) % 3)
        for b in range(BPT - 3, BPT):
            scatter_wait(b, b % 3)
        plsc.subcore_barrier()
        for t in range(NR // K):
            pltpu.sync_copy(acc_sh.at[pl.ds(s * NR + t * K, K)],
                            out_hbm.at[c, pl.ds(s * NR + t * K, K)])
        plsc.subcore_barrier()


def _tc_prescale_body(degp_ref, x_ref, dis_ref, ya_ref, yb_ref):
    deg = degp_ref[0] + degp_ref[1]
    pos = deg > 0.0
    dis = jnp.where(pos, lax.rsqrt(jnp.where(pos, deg, 1.0)), 0.0)
    dis_ref[...] = dis
    ya_ref[...] = (dis * x_ref[:, :DH]).astype(jnp.bfloat16)
    yb_ref[...] = (dis * x_ref[:, DH:]).astype(jnp.bfloat16)


def _tc_mid_body(dis_ref, pa_ref, pb_ref, ya_ref, yb_ref):
    d2 = dis_ref[...] * dis_ref[...]
    ya_ref[...] = (d2 * (pa_ref[0] + pa_ref[1])).astype(jnp.bfloat16)
    yb_ref[...] = (d2 * (pb_ref[0] + pb_ref[1])).astype(jnp.bfloat16)


def _tc_final_body(dis_ref, qa_ref, qb_ref, h_ref):
    dis = dis_ref[...]
    h_ref[:, :DH] = dis * (qa_ref[0] + qa_ref[1])
    h_ref[:, DH:] = dis * (qb_ref[0] + qb_ref[1])


_tc_prescale = pl.pallas_call(
    _tc_prescale_body,
    out_shape=(
        jax.ShapeDtypeStruct((N_PAD, 1), jnp.float32),
        jax.ShapeDtypeStruct((N_PAD, DH), jnp.bfloat16),
        jax.ShapeDtypeStruct((N_PAD, DH), jnp.bfloat16),
    ),
)

_tc_mid = pl.pallas_call(
    _tc_mid_body,
    out_shape=(
        jax.ShapeDtypeStruct((N_PAD, DH), jnp.bfloat16),
        jax.ShapeDtypeStruct((N_PAD, DH), jnp.bfloat16),
    ),
)

_COLPERM = tuple(
    32 * (q // 32) + ((q % 32) >> 1) + L * ((q % 32) & 1) for q in range(DH)
)

_tc_final = pl.pallas_call(
    _tc_final_body,
    out_shape=jax.ShapeDtypeStruct((N_PAD, D), jnp.float32),
)


@jax.jit
def kernel(x, edge_index, edge_weight):
    src = edge_index[0].astype(jnp.int32)
    dst = edge_index[1].astype(jnp.int32)
    srcp = jnp.pad(src, (0, EP - N_EDGES)).reshape(NW * BPT, K)
    dstp = jnp.pad(dst, (0, EP - N_EDGES)).reshape(NW * BPT, K)
    wp = jnp.pad(edge_weight, (0, EP - N_EDGES)).reshape(NW * BPT, K)
    xp = jnp.pad(x, ((0, N_PAD - N_NODES), (0, 0)))

    perm = jnp.array(_COLPERM, jnp.int32)
    degp = _sc_degree(dstp, wp)
    dis, ya, yb = _tc_prescale(degp[:, :, None], xp)
    pa, pb = _sc_edge_pass(ya[:, perm], yb[:, perm], srcp, dstp, wp)
    y2a, y2b = _tc_mid(dis, pa, pb)
    qa, qb = _sc_edge_pass(y2a[:, perm], y2b[:, perm], srcp, dstp, wp)
    h = _tc_final(dis, qa, qb)
    return h[:N_NODES]


# speedup vs baseline: 1.0416x; 1.0416x over previous
import functools

import jax
import jax.numpy as jnp
from jax import lax
from jax.experimental import pallas as pl
from jax.experimental.pallas import tpu as pltpu
from jax.experimental.pallas import tpu_sc as plsc

N_NODES = 10000
N_EDGES = 320000
D = 128
DH = 64

NC, NS, L = 2, 16, 16
NW = NC * NS
K = 128
BPT = 80
EP = NW * BPT * K
N_PAD = 10240
NR = N_PAD // NS


def _sc_mesh():
    return plsc.VectorSubcoreMesh(core_axis_name="c", subcore_axis_name="s")


@functools.partial(
    pl.kernel,
    out_type=jax.ShapeDtypeStruct((NC, N_PAD), jnp.float32),
    mesh=_sc_mesh(),
    scratch_types=[
        pltpu.VMEM((BPT, K), jnp.int32),
        pltpu.VMEM((BPT, K), jnp.float32),
        pltpu.VMEM((NR,), jnp.float32),
        pltpu.VMEM_SHARED((N_PAD,), jnp.float32),
    ],
)
def _sc_degree(dst_hbm, w_hbm, out_hbm, didx_all, w_all, z_v, deg_sh):
    c = lax.axis_index("c")
    s = lax.axis_index("s")
    wid = s * NC + c

    pltpu.sync_copy(dst_hbm.at[pl.ds(wid * BPT, BPT)], didx_all)
    pltpu.sync_copy(w_hbm.at[pl.ds(wid * BPT, BPT)], w_all)

    zero = jnp.zeros((L,), jnp.float32)
    for j in range(NR // L):
        z_v[pl.ds(j * L, L)] = zero
    pltpu.sync_copy(z_v, deg_sh.at[pl.ds(s * NR, NR)])
    plsc.subcore_barrier()

    def body(b, carry):
        pltpu.sync_copy(w_all.at[b], deg_sh.at[didx_all.at[b]], add=True)
        return carry

    lax.fori_loop(0, BPT, body, 0)
    plsc.subcore_barrier()
    pltpu.sync_copy(deg_sh.at[pl.ds(s * NR, NR)], out_hbm.at[c, pl.ds(s * NR, NR)])


@functools.partial(
    pl.kernel,
    out_type=jax.ShapeDtypeStruct((NC, N_PAD, DH), jnp.float32),
    mesh=_sc_mesh(),
    scratch_types=[
        pltpu.VMEM((BPT, K), jnp.int32),
        pltpu.VMEM((BPT, K), jnp.int32),
        pltpu.VMEM((BPT, K), jnp.float32),
        pltpu.VMEM((K, DH), jnp.bfloat16),
        pltpu.VMEM((K, DH), jnp.bfloat16),
        pltpu.VMEM((K, DH), jnp.bfloat16),
        pltpu.VMEM((K, DH), jnp.float32),
        pltpu.VMEM((K, DH), jnp.float32),
        pltpu.VMEM((K, DH), jnp.float32),
        pltpu.VMEM_SHARED((N_PAD, DH), jnp.bfloat16),
        pltpu.VMEM_SHARED((N_PAD, DH), jnp.float32),
        pltpu.SemaphoreType.DMA,
        pltpu.SemaphoreType.DMA,
        pltpu.SemaphoreType.DMA,
        pltpu.SemaphoreType.DMA,
        pltpu.SemaphoreType.DMA,
        pltpu.SemaphoreType.DMA,
    ],
    compiler_params=pltpu.CompilerParams(use_tc_tiling_on_sc=False,
                                         needs_layout_passes=False),
)
def _sc_edge_pass(y_hbm, src_hbm, dst_hbm, w_hbm, out_hbm,
                  sidx_all, didx_all, w_all, rb0, rb1, rb2, rf0, rf1, rf2,
                  y_sh, acc_sh, g0, g1, g2, s0, s1, s2):
    c = lax.axis_index("c")
    s = lax.axis_index("s")
    wid = s * NC + c
    rbf = (rb0, rb1, rb2)
    rf = (rf0, rf1, rf2)
    gsem = (g0, g1, g2)
    ssem = (s0, s1, s2)

    pltpu.sync_copy(src_hbm.at[pl.ds(wid * BPT, BPT)], sidx_all)
    pltpu.sync_copy(dst_hbm.at[pl.ds(wid * BPT, BPT)], didx_all)
    pltpu.sync_copy(w_hbm.at[pl.ds(wid * BPT, BPT)], w_all)

    pltpu.sync_copy(y_hbm.at[pl.ds(s * NR, NR)], y_sh.at[pl.ds(s * NR, NR)])

    zero = jnp.zeros((L,), jnp.float32)

    def zero_rows(r, carry):
        for j in range(DH // L):
            rf0[r, pl.ds(j * L, L)] = zero
        return carry

    lax.fori_loop(0, K, zero_rows, 0)
    for t in range(NR // K):
        pltpu.sync_copy(rf0, acc_sh.at[pl.ds(s * NR + t * K, K)])
    plsc.subcore_barrier()

    def gather(b, p):
        pltpu.async_copy(y_sh.at[sidx_all.at[b]], rbf[p], gsem[p])

    def gather_wait(b, p):
        pltpu.make_async_copy(y_sh.at[sidx_all.at[b]], rbf[p], gsem[p]).wait()

    def scatter(b, p):
        pltpu.async_copy(rf[p], acc_sh.at[didx_all.at[b]], ssem[p], add=True)

    def scatter_wait(b, p):
        pltpu.make_async_copy(rf[p], acc_sh.at[didx_all.at[b]], ssem[p]).wait()

    himask = jnp.full((L,), -65536, jnp.int32)

    def scale(b, p):
        def g_body(g, carry):
            wg = w_all[b, pl.ds(g * L, L)]
            for i in range(L):
                wk = wg[i]
                k = g * L + i
                for j in range(DH // (2 * L)):
                    packed = plsc.bitcast(rbf[p][k, pl.ds(j * 2 * L, 2 * L)],
                                          jnp.int32)
                    lo = plsc.bitcast(lax.shift_left(packed, 16), jnp.float32)
                    hi = plsc.bitcast(lax.bitwise_and(packed, himask),
                                      jnp.float32)
                    rf[p][k, pl.ds(j * 2 * L, L)] = lo * wk
                    rf[p][k, pl.ds(j * 2 * L + L, L)] = hi * wk
            return carry

        lax.fori_loop(0, K // L, g_body, 0)

    gather(0, 0)
    gather(1, 1)

    def body(i, carry):
        for u in range(3):
            b = 3 * i + u
            p = u
            gather_wait(b, p)

            @pl.when(b >= 3)
            def _():
                scatter_wait(b - 3, p)

            scale(b, p)
            scatter(b, p)

            @pl.when(b + 2 < BPT)
            def _():
                gather(b + 2, (u + 2) % 3)
        return carry

    n_main = BPT // 3 - 1
    lax.fori_loop(0, n_main, body, 0)

    for b in range(3 * n_main, BPT):
        p = b % 3
        gather_wait(b, p)
        scatter_wait(b - 3, p)
        scale(b, p)
        scatter(b, p)
        if b + 2 < BPT:
            gather(b + 2, (b + 2) % 3)
    for b in range(BPT - 3, BPT):
        scatter_wait(b, b % 3)
    plsc.subcore_barrier()
    for t in range(NR // K):
        pltpu.sync_copy(acc_sh.at[pl.ds(s * NR + t * K, K)],
                        out_hbm.at[c, pl.ds(s * NR + t * K, K)])


def _tc_prescale_body(degp_ref, x_ref, dis_ref, ya_ref, yb_ref):
    deg = degp_ref[0] + degp_ref[1]
    pos = deg > 0.0
    dis = jnp.where(pos, lax.rsqrt(jnp.where(pos, deg, 1.0)), 0.0)
    dis_ref[...] = dis
    cp = jnp.broadcast_to(_colperm_vec()[None, :], (N_PAD, DH))
    ya_ref[...] = jnp.take_along_axis(dis * x_ref[:, :DH], cp,
                                      axis=1).astype(jnp.bfloat16)
    yb_ref[...] = jnp.take_along_axis(dis * x_ref[:, DH:], cp,
                                      axis=1).astype(jnp.bfloat16)


def _colperm_vec():
    q = lax.iota(jnp.int32, DH)
    g = q // 32
    r = q % 32
    return g * 32 + r // 2 + (r % 2) * L


def _tc_mid_body(dis_ref, pa_ref, pb_ref, ya_ref, yb_ref):
    d2 = dis_ref[...] * dis_ref[...]
    cp = jnp.broadcast_to(_colperm_vec()[None, :], (N_PAD, DH))
    ya_ref[...] = jnp.take_along_axis(d2 * (pa_ref[0] + pa_ref[1]), cp,
                                      axis=1).astype(jnp.bfloat16)
    yb_ref[...] = jnp.take_along_axis(d2 * (pb_ref[0] + pb_ref[1]), cp,
                                      axis=1).astype(jnp.bfloat16)


def _tc_final_body(dis_ref, qa_ref, qb_ref, h_ref):
    dis = dis_ref[...]
    h_ref[:, :DH] = dis * (qa_ref[0] + qa_ref[1])
    h_ref[:, DH:] = dis * (qb_ref[0] + qb_ref[1])


_tc_prescale = pl.pallas_call(
    _tc_prescale_body,
    out_shape=(
        jax.ShapeDtypeStruct((N_PAD, 1), jnp.float32),
        jax.ShapeDtypeStruct((N_PAD, DH), jnp.bfloat16),
        jax.ShapeDtypeStruct((N_PAD, DH), jnp.bfloat16),
    ),
)

_tc_mid = pl.pallas_call(
    _tc_mid_body,
    out_shape=(
        jax.ShapeDtypeStruct((N_PAD, DH), jnp.bfloat16),
        jax.ShapeDtypeStruct((N_PAD, DH), jnp.bfloat16),
    ),
)

_COLPERM = tuple(
    32 * (q // 32) + ((q % 32) >> 1) + L * ((q % 32) & 1) for q in range(DH)
)

_tc_final = pl.pallas_call(
    _tc_final_body,
    out_shape=jax.ShapeDtypeStruct((N_PAD, D), jnp.float32),
)


@jax.jit
def kernel(x, edge_index, edge_weight):
    src = edge_index[0].astype(jnp.int32)
    dst = edge_index[1].astype(jnp.int32)
    srcp = jnp.pad(src, (0, EP - N_EDGES)).reshape(NW * BPT, K)
    dstp = jnp.pad(dst, (0, EP - N_EDGES)).reshape(NW * BPT, K)
    wp = jnp.pad(edge_weight, (0, EP - N_EDGES)).reshape(NW * BPT, K)
    xp = jnp.pad(x, ((0, N_PAD - N_NODES), (0, 0)))

    degp = _sc_degree(dstp, wp)
    dis, ya, yb = _tc_prescale(degp[:, :, None], xp)
    pa = _sc_edge_pass(ya, srcp, dstp, wp)
    pb = _sc_edge_pass(yb, srcp, dstp, wp)
    y2a, y2b = _tc_mid(dis, pa, pb)
    qa = _sc_edge_pass(y2a, srcp, dstp, wp)
    qb = _sc_edge_pass(y2b, srcp, dstp, wp)
    h = _tc_final(dis, qa, qb)
    return h[:N_NODES]


# speedup vs baseline: 1.0469x; 1.0051x over previous
import functools

import jax
import jax.numpy as jnp
from jax import lax
from jax.experimental import pallas as pl
from jax.experimental.pallas import tpu as pltpu
from jax.experimental.pallas import tpu_sc as plsc

N_NODES = 10000
N_EDGES = 320000
D = 128
DH = 64

NC, NS, L = 2, 16, 16
NW = NC * NS
K = 128
BPT = 80
EP = NW * BPT * K
N_PAD = 10240
NR = N_PAD // NS


def _sc_mesh():
    return plsc.VectorSubcoreMesh(core_axis_name="c", subcore_axis_name="s")


@functools.partial(
    pl.kernel,
    out_type=jax.ShapeDtypeStruct((NC, N_PAD), jnp.float32),
    mesh=_sc_mesh(),
    scratch_types=[
        pltpu.VMEM((BPT, K), jnp.int32),
        pltpu.VMEM((BPT, K), jnp.float32),
        pltpu.VMEM((NR,), jnp.float32),
        pltpu.VMEM_SHARED((N_PAD,), jnp.float32),
    ],
)
def _sc_degree(dst_hbm, w_hbm, out_hbm, didx_all, w_all, z_v, deg_sh):
    c = lax.axis_index("c")
    s = lax.axis_index("s")
    wid = s * NC + c

    pltpu.sync_copy(dst_hbm.at[pl.ds(wid * BPT, BPT)], didx_all)
    pltpu.sync_copy(w_hbm.at[pl.ds(wid * BPT, BPT)], w_all)

    zero = jnp.zeros((L,), jnp.float32)
    for j in range(NR // L):
        z_v[pl.ds(j * L, L)] = zero
    pltpu.sync_copy(z_v, deg_sh.at[pl.ds(s * NR, NR)])
    plsc.subcore_barrier()

    def body(b, carry):
        pltpu.sync_copy(w_all.at[b], deg_sh.at[didx_all.at[b]], add=True)
        return carry

    lax.fori_loop(0, BPT, body, 0)
    plsc.subcore_barrier()
    pltpu.sync_copy(deg_sh.at[pl.ds(s * NR, NR)], out_hbm.at[c, pl.ds(s * NR, NR)])


@functools.partial(
    pl.kernel,
    out_type=jax.ShapeDtypeStruct((NC, N_PAD, DH), jnp.float32),
    mesh=_sc_mesh(),
    scratch_types=[
        pltpu.VMEM((BPT, K), jnp.int32),
        pltpu.VMEM((BPT, K), jnp.int32),
        pltpu.VMEM((BPT, K), jnp.float32),
        pltpu.VMEM((K, DH), jnp.bfloat16),
        pltpu.VMEM((K, DH), jnp.bfloat16),
        pltpu.VMEM((K, DH), jnp.bfloat16),
        pltpu.VMEM((K, DH), jnp.float32),
        pltpu.VMEM((K, DH), jnp.float32),
        pltpu.VMEM((K, DH), jnp.float32),
        pltpu.VMEM_SHARED((N_PAD, DH), jnp.bfloat16),
        pltpu.VMEM_SHARED((N_PAD, DH), jnp.float32),
        pltpu.SemaphoreType.DMA,
        pltpu.SemaphoreType.DMA,
        pltpu.SemaphoreType.DMA,
        pltpu.SemaphoreType.DMA,
        pltpu.SemaphoreType.DMA,
        pltpu.SemaphoreType.DMA,
    ],
    compiler_params=pltpu.CompilerParams(use_tc_tiling_on_sc=False,
                                         needs_layout_passes=False),
)
def _sc_edge_pass(y_hbm, src_hbm, dst_hbm, w_hbm, out_hbm,
                  sidx_all, didx_all, w_all, rb0, rb1, rb2, rf0, rf1, rf2,
                  y_sh, acc_sh, g0, g1, g2, s0, s1, s2):
    c = lax.axis_index("c")
    s = lax.axis_index("s")
    wid = s * NC + c
    rbf = (rb0, rb1, rb2)
    rf = (rf0, rf1, rf2)
    gsem = (g0, g1, g2)
    ssem = (s0, s1, s2)

    pltpu.sync_copy(src_hbm.at[pl.ds(wid * BPT, BPT)], sidx_all)
    pltpu.sync_copy(dst_hbm.at[pl.ds(wid * BPT, BPT)], didx_all)
    pltpu.sync_copy(w_hbm.at[pl.ds(wid * BPT, BPT)], w_all)

    pltpu.sync_copy(y_hbm.at[pl.ds(s * NR, NR)], y_sh.at[pl.ds(s * NR, NR)])

    zero = jnp.zeros((L,), jnp.float32)

    def zero_rows(r, carry):
        for j in range(DH // L):
            rf0[r, pl.ds(j * L, L)] = zero
        return carry

    lax.fori_loop(0, K, zero_rows, 0)
    for t in range(NR // K):
        pltpu.sync_copy(rf0, acc_sh.at[pl.ds(s * NR + t * K, K)])
    plsc.subcore_barrier()

    def gather(b, p):
        pltpu.async_copy(y_sh.at[sidx_all.at[b]], rbf[p], gsem[p])

    def gather_wait(b, p):
        pltpu.make_async_copy(y_sh.at[sidx_all.at[b]], rbf[p], gsem[p]).wait()

    def scatter(b, p):
        pltpu.async_copy(rf[p], acc_sh.at[didx_all.at[b]], ssem[p], add=True)

    def scatter_wait(b, p):
        pltpu.make_async_copy(rf[p], acc_sh.at[didx_all.at[b]], ssem[p]).wait()

    himask = jnp.full((L,), -65536, jnp.int32)

    def scale(b, p):
        def g_body(g, carry):
            wg = w_all[b, pl.ds(g * L, L)]
            for i in range(L):
                wk = wg[i]
                k = g * L + i
                for j in range(DH // (2 * L)):
                    packed = plsc.bitcast(rbf[p][k, pl.ds(j * 2 * L, 2 * L)],
                                          jnp.int32)
                    lo = plsc.bitcast(lax.shift_left(packed, 16), jnp.float32)
                    hi = plsc.bitcast(lax.bitwise_and(packed, himask),
                                      jnp.float32)
                    rf[p][k, pl.ds(j * 2 * L, L)] = lo * wk
                    rf[p][k, pl.ds(j * 2 * L + L, L)] = hi * wk
            return carry

        lax.fori_loop(0, K // L, g_body, 0)

    gather(0, 0)
    gather(1, 1)

    def body(i, carry):
        for u in range(3):
            b = 3 * i + u
            p = u
            gather_wait(b, p)

            @pl.when(b >= 3)
            def _():
                scatter_wait(b - 3, p)

            scale(b, p)
            scatter(b, p)

            @pl.when(b + 2 < BPT)
            def _():
                gather(b + 2, (u + 2) % 3)
        return carry

    n_main = BPT // 3 - 1
    lax.fori_loop(0, n_main, body, 0)

    for b in range(3 * n_main, BPT):
        p = b % 3
        gather_wait(b, p)
        scatter_wait(b - 3, p)
        scale(b, p)
        scatter(b, p)
        if b + 2 < BPT:
            gather(b + 2, (b + 2) % 3)
    for b in range(BPT - 3, BPT):
        scatter_wait(b, b % 3)
    plsc.subcore_barrier()
    for t in range(NR // K):
        pltpu.sync_copy(acc_sh.at[pl.ds(s * NR + t * K, K)],
                        out_hbm.at[c, pl.ds(s * NR + t * K, K)])


def _tc_prescale_body(degp_ref, x_ref, dis_ref, ya_ref, yb_ref):
    deg = degp_ref[0] + degp_ref[1]
    pos = deg > 0.0
    dis = jnp.where(pos, lax.rsqrt(jnp.where(pos, deg, 1.0)), 0.0)
    dis_ref[...] = dis
    cp = jnp.broadcast_to(_colperm_vec()[None, :], (N_NODES, DH))
    dn = dis[:N_NODES]
    ya_ref[:N_NODES] = jnp.take_along_axis(dn * x_ref[:, :DH], cp,
                                           axis=1).astype(jnp.bfloat16)
    yb_ref[:N_NODES] = jnp.take_along_axis(dn * x_ref[:, DH:], cp,
                                           axis=1).astype(jnp.bfloat16)
    pad = jnp.zeros((N_PAD - N_NODES, DH), jnp.bfloat16)
    ya_ref[N_NODES:] = pad
    yb_ref[N_NODES:] = pad


def _colperm_vec():
    q = lax.iota(jnp.int32, DH)
    g = q // 32
    r = q % 32
    return g * 32 + r // 2 + (r % 2) * L


def _tc_mid_body(dis_ref, pa_ref, pb_ref, ya_ref, yb_ref):
    d2 = dis_ref[...] * dis_ref[...]
    cp = jnp.broadcast_to(_colperm_vec()[None, :], (N_PAD, DH))
    ya_ref[...] = jnp.take_along_axis(d2 * (pa_ref[0] + pa_ref[1]), cp,
                                      axis=1).astype(jnp.bfloat16)
    yb_ref[...] = jnp.take_along_axis(d2 * (pb_ref[0] + pb_ref[1]), cp,
                                      axis=1).astype(jnp.bfloat16)


def _tc_final_body(dis_ref, qa_ref, qb_ref, h_ref):
    dis = dis_ref[:N_NODES]
    h_ref[:, :DH] = dis * (qa_ref[0, :N_NODES] + qa_ref[1, :N_NODES])
    h_ref[:, DH:] = dis * (qb_ref[0, :N_NODES] + qb_ref[1, :N_NODES])


_tc_prescale = pl.pallas_call(
    _tc_prescale_body,
    out_shape=(
        jax.ShapeDtypeStruct((N_PAD, 1), jnp.float32),
        jax.ShapeDtypeStruct((N_PAD, DH), jnp.bfloat16),
        jax.ShapeDtypeStruct((N_PAD, DH), jnp.bfloat16),
    ),
)

_tc_mid = pl.pallas_call(
    _tc_mid_body,
    out_shape=(
        jax.ShapeDtypeStruct((N_PAD, DH), jnp.bfloat16),
        jax.ShapeDtypeStruct((N_PAD, DH), jnp.bfloat16),
    ),
)

_COLPERM = tuple(
    32 * (q // 32) + ((q % 32) >> 1) + L * ((q % 32) & 1) for q in range(DH)
)

_tc_final = pl.pallas_call(
    _tc_final_body,
    out_shape=jax.ShapeDtypeStruct((N_NODES, D), jnp.float32),
)


@jax.jit
def kernel(x, edge_index, edge_weight):
    src = edge_index[0].astype(jnp.int32)
    dst = edge_index[1].astype(jnp.int32)
    srcp = jnp.pad(src, (0, EP - N_EDGES)).reshape(NW * BPT, K)
    dstp = jnp.pad(dst, (0, EP - N_EDGES)).reshape(NW * BPT, K)
    wp = jnp.pad(edge_weight, (0, EP - N_EDGES)).reshape(NW * BPT, K)

    degp = _sc_degree(dstp, wp)
    dis, ya, yb = _tc_prescale(degp[:, :, None], x)
    pa = _sc_edge_pass(ya, srcp, dstp, wp)
    pb = _sc_edge_pass(yb, srcp, dstp, wp)
    y2a, y2b = _tc_mid(dis, pa, pb)
    qa = _sc_edge_pass(y2a, srcp, dstp, wp)
    qb = _sc_edge_pass(y2b, srcp, dstp, wp)
    return _tc_final(dis, qa, qb)
